# Initial kernel scaffold; baseline (speedup 1.0000x reference)
import jax
import jax.numpy as jnp
from jax.experimental import pallas as pl


def kernel(features, edge_index, edge_type, norm, bases, w_comp, layer_bias, mlp_w, mlp_b):
    raise NotImplementedError("write your pallas kernel here")


# speedup vs baseline: 106.8863x; 106.8863x over previous
import functools

import jax
import jax.numpy as jnp
from jax import lax
from jax.experimental import pallas as pl
from jax.experimental.pallas import tpu as pltpu
from jax.experimental.pallas import tpu_sc as plsc

_N = 100000
_E = 6400000
_R = 16
_H = 2
_NH = 4

_NP = 102400
_TS = _NP // 16
_CW = 2048
_NCH = _E // _CW
_KMAX = -(-_NCH // 32)


def _make_layer_kernel(mode):
    out_type = [jax.ShapeDtypeStruct((4, _NP), jnp.float32)]
    if mode == 2:
        out_type.append(jax.ShapeDtypeStruct((2, _NP), jnp.float32))

    scratch = [
        pltpu.VMEM_SHARED((_NP,), jnp.float32),
        pltpu.VMEM_SHARED((_NP,), jnp.float32),
        pltpu.VMEM_SHARED((_NP,), jnp.float32),
        pltpu.VMEM_SHARED((_NP,), jnp.float32),
        pltpu.VMEM((_CW,), jnp.int32),
        pltpu.VMEM((_CW,), jnp.int32),
        pltpu.VMEM((_CW,), jnp.int32),
        pltpu.VMEM((_CW,), jnp.float32),
        pltpu.VMEM((_CW,), jnp.float32),
        pltpu.VMEM((_CW,), jnp.float32),
        pltpu.VMEM((_CW,), jnp.float32),
        pltpu.VMEM((_CW,), jnp.float32),
        pltpu.VMEM((4, 16), jnp.float32),
        pltpu.VMEM((_TS,), jnp.float32),
        pltpu.VMEM((_TS,), jnp.float32),
        pltpu.VMEM((_TS,), jnp.float32),
        pltpu.SemaphoreType.DMA,
    ]
    if mode in (1, 2, 3):
        scratch += [
            pltpu.VMEM((2, 16), jnp.float32),
            pltpu.VMEM((_TS,), jnp.float32),
            pltpu.VMEM((_TS,), jnp.float32),
            pltpu.VMEM((_TS,), jnp.float32),
            pltpu.VMEM((_TS,), jnp.float32),
        ]
    if mode in (0, 2):
        scratch += [pltpu.VMEM((2 * _TS,), jnp.float32)]

    mesh = plsc.VectorSubcoreMesh(core_axis_name="c", subcore_axis_name="s")

    def body(*refs):
        it = iter(refs)
        src_h = next(it)
        dst_h = next(it)
        type_h = next(it)
        norm_h = next(it)
        wtab_h = next(it)
        if mode in (1, 2, 3):
            hp_h = next(it)
            bias_h = next(it)
        if mode in (0, 2):
            xf_h = next(it)
        hout = next(it)
        if mode == 2:
            xout = next(it)
        sc = dict(
            x0s=next(it), x1s=next(it), h0s=next(it), h1s=next(it),
            srcbuf=next(it), dstbuf=next(it), typebuf=next(it),
            normbuf=next(it), xs0=next(it), xs1=next(it), m0=next(it),
            m1=next(it), wbuf=next(it), xb0=next(it), xb1=next(it),
            zb=next(it), sem=next(it))
        if mode in (1, 2, 3):
            sc.update(bbuf=next(it), hp0=next(it), hp1=next(it),
                      hp2=next(it), hp3=next(it))
        if mode in (0, 2):
            sc.update(fbuf=next(it))

        c = lax.axis_index("c")
        s = lax.axis_index("s")
        wid = s * 2 + c
        nbase = s * _TS
        iot = lax.iota(jnp.int32, 16)

        pltpu.sync_copy(wtab_h, sc["wbuf"])
        if mode in (1, 2, 3):
            pltpu.sync_copy(bias_h, sc["bbuf"])
            pltpu.sync_copy(hp_h.at[0, pl.ds(nbase, _TS)], sc["hp0"])
            pltpu.sync_copy(hp_h.at[1, pl.ds(nbase, _TS)], sc["hp1"])
            pltpu.sync_copy(hp_h.at[2, pl.ds(nbase, _TS)], sc["hp2"])
            pltpu.sync_copy(hp_h.at[3, pl.ds(nbase, _TS)], sc["hp3"])
            b0v = sc["bbuf"][0, :]
            b1v = sc["bbuf"][1, :]
        if mode in (0, 2):
            pltpu.sync_copy(xf_h.at[pl.ds(s * 2 * _TS, 2 * _TS)], sc["fbuf"])

        zero = jnp.zeros((16,), jnp.float32)

        def pro_body(j, _):
            off = j * 16
            if mode in (0, 2):
                f0 = plsc.load_gather(sc["fbuf"], [(off + iot) * 2])
                f1 = plsc.load_gather(sc["fbuf"], [(off + iot) * 2 + 1])
            if mode == 0:
                x0v, x1v = f0, f1
            else:
                x0v = sc["hp0"][pl.ds(off, 16)] + sc["hp2"][pl.ds(off, 16)] + b0v
                x1v = sc["hp1"][pl.ds(off, 16)] + sc["hp3"][pl.ds(off, 16)] + b1v
                x0v = jnp.maximum(x0v, 0.0)
                x1v = jnp.maximum(x1v, 0.0)
                if mode == 2:
                    x0v = x0v + f0
                    x1v = x1v + f1
            sc["xb0"][pl.ds(off, 16)] = x0v
            sc["xb1"][pl.ds(off, 16)] = x1v
            sc["zb"][pl.ds(off, 16)] = zero
            return 0

        lax.fori_loop(0, _TS // 16, pro_body, 0)

        pltpu.sync_copy(sc["xb0"], sc["x0s"].at[pl.ds(nbase, _TS)])
        pltpu.sync_copy(sc["xb1"], sc["x1s"].at[pl.ds(nbase, _TS)])
        pltpu.sync_copy(sc["zb"], sc["h0s"].at[pl.ds(nbase, _TS)])
        pltpu.sync_copy(sc["zb"], sc["h1s"].at[pl.ds(nbase, _TS)])
        if mode == 2:
            @pl.when(c == 0)
            def _():
                pltpu.sync_copy(sc["xb0"], xout.at[0, pl.ds(nbase, _TS)])
                pltpu.sync_copy(sc["xb1"], xout.at[1, pl.ds(nbase, _TS)])
        plsc.subcore_barrier()

        def chunk_body(k, _):
            cidx = wid + 32 * k

            @pl.when(cidx < _NCH)
            def _():
                pltpu.sync_copy(src_h.at[cidx], sc["srcbuf"])
                pltpu.sync_copy(dst_h.at[cidx], sc["dstbuf"])
                pltpu.sync_copy(type_h.at[cidx], sc["typebuf"])
                pltpu.sync_copy(norm_h.at[cidx], sc["normbuf"])
                g0 = pltpu.async_copy(sc["x0s"].at[sc["srcbuf"]], sc["xs0"],
                                      sc["sem"])
                g1 = pltpu.async_copy(sc["x1s"].at[sc["srcbuf"]], sc["xs1"],
                                      sc["sem"])
                g0.wait()
                g1.wait()

                def row_body(jj, _):
                    for sub in range(4):
                        off = jj * 64 + sub * 16
                        t = sc["typebuf"][pl.ds(off, 16)]
                        nv = sc["normbuf"][pl.ds(off, 16)]
                        a0 = sc["xs0"][pl.ds(off, 16)]
                        a1 = sc["xs1"][pl.ds(off, 16)]
                        w00 = plsc.load_gather(sc["wbuf"].at[0], [t])
                        w10 = plsc.load_gather(sc["wbuf"].at[1], [t])
                        w01 = plsc.load_gather(sc["wbuf"].at[2], [t])
                        w11 = plsc.load_gather(sc["wbuf"].at[3], [t])
                        na0 = a0 * nv
                        na1 = a1 * nv
                        sc["m0"][pl.ds(off, 16)] = na0 * w00 + na1 * w10
                        sc["m1"][pl.ds(off, 16)] = na0 * w01 + na1 * w11
                    return 0

                lax.fori_loop(0, _CW // 64, row_body, 0)
                s0 = pltpu.async_copy(sc["m0"], sc["h0s"].at[sc["dstbuf"]],
                                      sc["sem"], add=True)
                s1 = pltpu.async_copy(sc["m1"], sc["h1s"].at[sc["dstbuf"]],
                                      sc["sem"], add=True)
                s0.wait()
                s1.wait()
            return 0

        lax.fori_loop(0, _KMAX, chunk_body, 0)
        plsc.subcore_barrier()

        @pl.when(c == 0)
        def _():
            pltpu.sync_copy(sc["h0s"].at[pl.ds(nbase, _TS)],
                            hout.at[0, pl.ds(nbase, _TS)])
            pltpu.sync_copy(sc["h1s"].at[pl.ds(nbase, _TS)],
                            hout.at[1, pl.ds(nbase, _TS)])

        @pl.when(c == 1)
        def _():
            pltpu.sync_copy(sc["h0s"].at[pl.ds(nbase, _TS)],
                            hout.at[2, pl.ds(nbase, _TS)])
            pltpu.sync_copy(sc["h1s"].at[pl.ds(nbase, _TS)],
                            hout.at[3, pl.ds(nbase, _TS)])

    return pl.kernel(body, out_type=tuple(out_type) if mode == 2
                     else out_type[0],
                     compiler_params=pltpu.CompilerParams(
                         needs_layout_passes=False),
                     mesh=mesh, scratch_types=scratch)


def _mlp_body(hp_ref, x2_ref, wc_ref, b3_ref, mb_ref, out_ref):
    x40 = hp_ref[0, :] + hp_ref[2, :] + b3_ref[0, 0] + x2_ref[0, :]
    x41 = hp_ref[1, :] + hp_ref[3, :] + b3_ref[0, 1] + x2_ref[1, :]
    acc = jnp.sum(x40 * wc_ref[0, :]) + jnp.sum(x41 * wc_ref[1, :])
    z = acc + mb_ref[0, 0]
    out_ref[0, 0] = 1.0 / (1.0 + jnp.exp(-z))


def kernel(features, edge_index, edge_type, norm, bases, w_comp, layer_bias,
           mlp_w, mlp_b):
    src3 = edge_index[0].reshape(_NCH, _CW)
    dst3 = edge_index[1].reshape(_NCH, _CW)
    type2 = edge_type.reshape(_NCH, _CW)
    norm2 = norm.reshape(_NCH, _CW)
    xf = jnp.pad(features.reshape(-1), (0, 2 * _NP - 2 * _N))

    W = jnp.einsum("lrb,lbio->lrio", w_comp, bases)
    wtabs = jnp.stack(
        [W[:, :, 0, 0], W[:, :, 1, 0], W[:, :, 0, 1], W[:, :, 1, 1]], axis=1
    )
    biases = jnp.broadcast_to(layer_bias[:, :, None], (_NH, _H, 16))

    k0 = _make_layer_kernel(0)
    k1 = _make_layer_kernel(1)
    k2 = _make_layer_kernel(2)
    k3 = _make_layer_kernel(3)

    h0 = k0(src3, dst3, type2, norm2, wtabs[0], xf)
    h1 = k1(src3, dst3, type2, norm2, wtabs[1], h0, biases[0])
    h2, x2 = k2(src3, dst3, type2, norm2, wtabs[2], h1, biases[1], xf)
    h3 = k3(src3, dst3, type2, norm2, wtabs[3], h2, biases[2])

    wcols = jnp.pad(mlp_w.reshape(_N, _H).T, ((0, 0), (0, _NP - _N)))
    b3 = layer_bias[3].reshape(1, _H)
    mb = mlp_b.reshape(1, 1)
    out = pl.pallas_call(
        _mlp_body,
        out_shape=jax.ShapeDtypeStruct((1, 1), jnp.float32),
        in_specs=[pl.BlockSpec(memory_space=pltpu.VMEM)] * 3
        + [pl.BlockSpec(memory_space=pltpu.SMEM)] * 2,
        out_specs=pl.BlockSpec(memory_space=pltpu.SMEM),
    )(h3, x2, wcols, b3, mb)
    return out.reshape(1, 1)


# speedup vs baseline: 118.0714x; 1.1046x over previous
import functools

import jax
import jax.numpy as jnp
from jax import lax
from jax.experimental import pallas as pl
from jax.experimental.pallas import tpu as pltpu
from jax.experimental.pallas import tpu_sc as plsc

_N = 100000
_E = 6400000
_R = 16
_H = 2
_NH = 4

_NP = 102400
_TS = _NP // 16
_CW = 2048
_NCH = _E // _CW
_KMAX = -(-_NCH // 32)


def _make_layer_kernel(mode):
    out_type = [jax.ShapeDtypeStruct((4, _NP), jnp.float32)]
    if mode == 2:
        out_type.append(jax.ShapeDtypeStruct((2, _NP), jnp.float32))

    scratch = [
        pltpu.VMEM_SHARED((_NP,), jnp.float32),
        pltpu.VMEM_SHARED((_NP,), jnp.float32),
        pltpu.VMEM_SHARED((_NP,), jnp.float32),
        pltpu.VMEM_SHARED((_NP,), jnp.float32),
        pltpu.VMEM((_CW,), jnp.int32),
        pltpu.VMEM((_CW,), jnp.int32),
        pltpu.VMEM((_CW,), jnp.int32),
        pltpu.VMEM((_CW,), jnp.int32),
        pltpu.VMEM((2, _CW), jnp.int32),
        pltpu.VMEM((2, _CW), jnp.int32),
        pltpu.VMEM((_CW,), jnp.float32),
        pltpu.VMEM((_CW,), jnp.float32),
        pltpu.VMEM((_CW,), jnp.float32),
        pltpu.VMEM((_CW,), jnp.float32),
        pltpu.VMEM((_CW,), jnp.float32),
        pltpu.VMEM((_CW,), jnp.float32),
        pltpu.VMEM((_CW,), jnp.float32),
        pltpu.VMEM((_CW,), jnp.float32),
        pltpu.VMEM((4, 16), jnp.float32),
        pltpu.VMEM((_TS,), jnp.float32),
        pltpu.VMEM((_TS,), jnp.float32),
        pltpu.VMEM((_TS,), jnp.float32),
        pltpu.SemaphoreType.DMA,
        pltpu.SemaphoreType.DMA,
        pltpu.SemaphoreType.DMA,
    ]
    if mode in (1, 2, 3):
        scratch += [
            pltpu.VMEM((2, 16), jnp.float32),
            pltpu.VMEM((_TS,), jnp.float32),
            pltpu.VMEM((_TS,), jnp.float32),
            pltpu.VMEM((_TS,), jnp.float32),
            pltpu.VMEM((_TS,), jnp.float32),
        ]
    if mode in (0, 2):
        scratch += [pltpu.VMEM((2 * _TS,), jnp.float32)]

    mesh = plsc.VectorSubcoreMesh(core_axis_name="c", subcore_axis_name="s")

    def body(*refs):
        it = iter(refs)
        src_h = next(it)
        dst_h = next(it)
        tn_h = next(it)
        wtab_h = next(it)
        if mode in (1, 2, 3):
            hp_h = next(it)
            bias_h = next(it)
        if mode in (0, 2):
            xf_h = next(it)
        hout = next(it)
        if mode == 2:
            xout = next(it)
        sc = dict(
            x0s=next(it), x1s=next(it), h0s=next(it), h1s=next(it),
            srcb0=next(it), srcb1=next(it), dstb0=next(it),
            dstb1=next(it), tnb0=next(it), tnb1=next(it),
            xs0a=next(it), xs1a=next(it), xs0b=next(it), xs1b=next(it),
            m0a=next(it), m1a=next(it), m0b=next(it), m1b=next(it),
            wbuf=next(it), xb0=next(it), xb1=next(it),
            zb=next(it), lsem=next(it), gsem=next(it), ssem=next(it))
        if mode in (1, 2, 3):
            sc.update(bbuf=next(it), hp0=next(it), hp1=next(it),
                      hp2=next(it), hp3=next(it))
        if mode in (0, 2):
            sc.update(fbuf=next(it))

        c = lax.axis_index("c")
        s = lax.axis_index("s")
        wid = s * 2 + c
        nbase = s * _TS
        iot = lax.iota(jnp.int32, 16)

        pltpu.sync_copy(wtab_h, sc["wbuf"])
        if mode in (1, 2, 3):
            pltpu.sync_copy(bias_h, sc["bbuf"])
            pltpu.sync_copy(hp_h.at[0, pl.ds(nbase, _TS)], sc["hp0"])
            pltpu.sync_copy(hp_h.at[1, pl.ds(nbase, _TS)], sc["hp1"])
            pltpu.sync_copy(hp_h.at[2, pl.ds(nbase, _TS)], sc["hp2"])
            pltpu.sync_copy(hp_h.at[3, pl.ds(nbase, _TS)], sc["hp3"])
            b0v = sc["bbuf"][0, :]
            b1v = sc["bbuf"][1, :]
        if mode in (0, 2):
            pltpu.sync_copy(xf_h.at[pl.ds(s * 2 * _TS, 2 * _TS)], sc["fbuf"])

        zero = jnp.zeros((16,), jnp.float32)

        def pro_body(j, _):
            off = j * 16
            if mode in (0, 2):
                f0 = plsc.load_gather(sc["fbuf"], [(off + iot) * 2])
                f1 = plsc.load_gather(sc["fbuf"], [(off + iot) * 2 + 1])
            if mode == 0:
                x0v, x1v = f0, f1
            else:
                x0v = sc["hp0"][pl.ds(off, 16)] + sc["hp2"][pl.ds(off, 16)] + b0v
                x1v = sc["hp1"][pl.ds(off, 16)] + sc["hp3"][pl.ds(off, 16)] + b1v
                x0v = jnp.maximum(x0v, 0.0)
                x1v = jnp.maximum(x1v, 0.0)
                if mode == 2:
                    x0v = x0v + f0
                    x1v = x1v + f1
            sc["xb0"][pl.ds(off, 16)] = x0v
            sc["xb1"][pl.ds(off, 16)] = x1v
            sc["zb"][pl.ds(off, 16)] = zero
            return 0

        lax.fori_loop(0, _TS // 16, pro_body, 0)

        pltpu.sync_copy(sc["xb0"], sc["x0s"].at[pl.ds(nbase, _TS)])
        pltpu.sync_copy(sc["xb1"], sc["x1s"].at[pl.ds(nbase, _TS)])
        pltpu.sync_copy(sc["zb"], sc["h0s"].at[pl.ds(nbase, _TS)])
        pltpu.sync_copy(sc["zb"], sc["h1s"].at[pl.ds(nbase, _TS)])
        if mode == 2:
            @pl.when(c == 0)
            def _():
                pltpu.sync_copy(sc["xb0"], xout.at[0, pl.ds(nbase, _TS)])
                pltpu.sync_copy(sc["xb1"], xout.at[1, pl.ds(nbase, _TS)])
        plsc.subcore_barrier()

        srcbs = (sc["srcb0"], sc["srcb1"])
        dstbs = (sc["dstb0"], sc["dstb1"])
        tnbs = (sc["tnb0"], sc["tnb1"])
        xs0s = (sc["xs0a"], sc["xs0b"])
        xs1s = (sc["xs1a"], sc["xs1b"])
        m0s = (sc["m0a"], sc["m0b"])
        m1s = (sc["m1a"], sc["m1b"])

        def issue_linear(cidx, b):
            pltpu.async_copy(src_h.at[cidx], srcbs[b], sc["lsem"])
            pltpu.async_copy(dst_h.at[cidx], dstbs[b], sc["lsem"])
            pltpu.async_copy(tn_h.at[cidx], tnbs[b], sc["lsem"])

        def wait_linear(cidx, b):
            pltpu.make_async_copy(src_h.at[cidx], srcbs[b], sc["lsem"]).wait()
            pltpu.make_async_copy(dst_h.at[cidx], dstbs[b], sc["lsem"]).wait()
            pltpu.make_async_copy(tn_h.at[cidx], tnbs[b], sc["lsem"]).wait()

        issue_linear(wid, 0)

        def chunk_pair(g, _):
            for b in (0, 1):
                k = g * 2 + b
                cidx = wid + 32 * k
                srcb, dstb, tnb = srcbs[b], dstbs[b], tnbs[b]
                xs0, xs1 = xs0s[b], xs1s[b]
                m0, m1 = m0s[b], m1s[b]

                @pl.when(cidx < _NCH)
                def _():
                    wait_linear(cidx, b)
                    g0 = pltpu.async_copy(sc["x0s"].at[srcb], xs0,
                                          sc["gsem"])
                    g1 = pltpu.async_copy(sc["x1s"].at[srcb], xs1,
                                          sc["gsem"])
                    ncidx = cidx + 32

                    @pl.when(ncidx < _NCH)
                    def _():
                        issue_linear(ncidx, 1 - b)

                    g0.wait()
                    g1.wait()

                    def row_body(jj, _):
                        for sub in range(4):
                            off = jj * 64 + sub * 16
                            t = tnb[0, pl.ds(off, 16)]
                            nv = plsc.bitcast(tnb[1, pl.ds(off, 16)],
                                              jnp.float32)
                            a0 = xs0[pl.ds(off, 16)]
                            a1 = xs1[pl.ds(off, 16)]
                            w00 = plsc.load_gather(sc["wbuf"].at[0], [t])
                            w10 = plsc.load_gather(sc["wbuf"].at[1], [t])
                            w01 = plsc.load_gather(sc["wbuf"].at[2], [t])
                            w11 = plsc.load_gather(sc["wbuf"].at[3], [t])
                            na0 = a0 * nv
                            na1 = a1 * nv
                            m0[pl.ds(off, 16)] = na0 * w00 + na1 * w10
                            m1[pl.ds(off, 16)] = na0 * w01 + na1 * w11
                        return 0

                    lax.fori_loop(0, _CW // 64, row_body, 0)
                    s0 = pltpu.async_copy(m0, sc["h0s"].at[dstb],
                                          sc["ssem"], add=True)
                    s1 = pltpu.async_copy(m1, sc["h1s"].at[dstb],
                                          sc["ssem"], add=True)
                    s0.wait()
                    s1.wait()
            return 0

        lax.fori_loop(0, _KMAX // 2, chunk_pair, 0)
        plsc.subcore_barrier()

        @pl.when(c == 0)
        def _():
            pltpu.sync_copy(sc["h0s"].at[pl.ds(nbase, _TS)],
                            hout.at[0, pl.ds(nbase, _TS)])
            pltpu.sync_copy(sc["h1s"].at[pl.ds(nbase, _TS)],
                            hout.at[1, pl.ds(nbase, _TS)])

        @pl.when(c == 1)
        def _():
            pltpu.sync_copy(sc["h0s"].at[pl.ds(nbase, _TS)],
                            hout.at[2, pl.ds(nbase, _TS)])
            pltpu.sync_copy(sc["h1s"].at[pl.ds(nbase, _TS)],
                            hout.at[3, pl.ds(nbase, _TS)])

    return pl.kernel(body, out_type=tuple(out_type) if mode == 2
                     else out_type[0],
                     compiler_params=pltpu.CompilerParams(
                         needs_layout_passes=False),
                     mesh=mesh, scratch_types=scratch)


def _mlp_body(hp_ref, x2_ref, wc_ref, b3_ref, mb_ref, out_ref):
    x40 = hp_ref[0, :] + hp_ref[2, :] + b3_ref[0, 0] + x2_ref[0, :]
    x41 = hp_ref[1, :] + hp_ref[3, :] + b3_ref[0, 1] + x2_ref[1, :]
    acc = jnp.sum(x40 * wc_ref[0, :]) + jnp.sum(x41 * wc_ref[1, :])
    z = acc + mb_ref[0, 0]
    out_ref[0, 0] = 1.0 / (1.0 + jnp.exp(-z))


def kernel(features, edge_index, edge_type, norm, bases, w_comp, layer_bias,
           mlp_w, mlp_b):
    src2 = edge_index[0].reshape(_NCH, _CW)
    dst2 = edge_index[1].reshape(_NCH, _CW)
    tn = jnp.stack(
        [edge_type.reshape(_NCH, _CW),
         jax.lax.bitcast_convert_type(norm.reshape(_NCH, _CW), jnp.int32)],
        axis=1)
    xf = jnp.pad(features.reshape(-1), (0, 2 * _NP - 2 * _N))

    W = jnp.einsum("lrb,lbio->lrio", w_comp, bases)
    wtabs = jnp.stack(
        [W[:, :, 0, 0], W[:, :, 1, 0], W[:, :, 0, 1], W[:, :, 1, 1]], axis=1
    )
    biases = jnp.broadcast_to(layer_bias[:, :, None], (_NH, _H, 16))

    k0 = _make_layer_kernel(0)
    k1 = _make_layer_kernel(1)
    k2 = _make_layer_kernel(2)
    k3 = _make_layer_kernel(3)

    h0 = k0(src2, dst2, tn, wtabs[0], xf)
    h1 = k1(src2, dst2, tn, wtabs[1], h0, biases[0])
    h2, x2 = k2(src2, dst2, tn, wtabs[2], h1, biases[1], xf)
    h3 = k3(src2, dst2, tn, wtabs[3], h2, biases[2])

    wcols = jnp.pad(mlp_w.reshape(_N, _H).T, ((0, 0), (0, _NP - _N)))
    b3 = layer_bias[3].reshape(1, _H)
    mb = mlp_b.reshape(1, 1)
    out = pl.pallas_call(
        _mlp_body,
        out_shape=jax.ShapeDtypeStruct((1, 1), jnp.float32),
        in_specs=[pl.BlockSpec(memory_space=pltpu.VMEM)] * 3
        + [pl.BlockSpec(memory_space=pltpu.SMEM)] * 2,
        out_specs=pl.BlockSpec(memory_space=pltpu.SMEM),
    )(h3, x2, wcols, b3, mb)
    return out.reshape(1, 1)


# speedup vs baseline: 175.6751x; 1.4879x over previous
import jax
import jax.numpy as jnp
from jax import lax
from jax.experimental import pallas as pl
from jax.experimental.pallas import tpu as pltpu
from jax.experimental.pallas import tpu_sc as plsc

_N = 100000
_E = 6400000
_R = 16
_H = 2
_NH = 4

_NP = 102400
_TS = _NP // 16
_CW = 5120
_NCH = _E // _CW
_KMAX = -(-_NCH // 32)
_NSEG = 4
_SEG = _TS // _NSEG


def _make_layer_kernel(mode):
    out_type = [jax.ShapeDtypeStruct((4 * _NP,), jnp.float32)]
    if mode == 2:
        out_type.append(jax.ShapeDtypeStruct((2 * _NP,), jnp.float32))

    scratch = [
        pltpu.VMEM_SHARED((_NP,), jnp.float32),
        pltpu.VMEM_SHARED((_NP,), jnp.float32),
        pltpu.VMEM_SHARED((_NP,), jnp.float32),
        pltpu.VMEM_SHARED((_NP,), jnp.float32),
        pltpu.VMEM((_CW,), jnp.int32),
        pltpu.VMEM((_CW,), jnp.int32),
        pltpu.VMEM((_CW,), jnp.int32),
        pltpu.VMEM((_CW,), jnp.int32),
        pltpu.VMEM((2, _CW), jnp.int32),
        pltpu.VMEM((2, _CW), jnp.int32),
        pltpu.VMEM((_CW,), jnp.float32),
        pltpu.VMEM((_CW,), jnp.float32),
        pltpu.VMEM((_CW,), jnp.float32),
        pltpu.VMEM((_CW,), jnp.float32),
        pltpu.VMEM((_CW,), jnp.float32),
        pltpu.VMEM((_CW,), jnp.float32),
        pltpu.VMEM((_CW,), jnp.float32),
        pltpu.VMEM((_CW,), jnp.float32),
        pltpu.VMEM((4, 16), jnp.float32),
        pltpu.VMEM((_SEG,), jnp.float32),
        pltpu.VMEM((_SEG,), jnp.float32),
        pltpu.VMEM((_SEG,), jnp.float32),
        pltpu.SemaphoreType.DMA,
        pltpu.SemaphoreType.DMA,
        pltpu.SemaphoreType.DMA,
    ]
    if mode in (1, 2, 3):
        scratch += [
            pltpu.VMEM((2, 16), jnp.float32),
            pltpu.VMEM((_SEG,), jnp.float32),
            pltpu.VMEM((_SEG,), jnp.float32),
            pltpu.VMEM((_SEG,), jnp.float32),
            pltpu.VMEM((_SEG,), jnp.float32),
        ]
    if mode in (0, 2):
        scratch += [pltpu.VMEM((2 * _SEG,), jnp.float32)]

    mesh = plsc.VectorSubcoreMesh(core_axis_name="c", subcore_axis_name="s")

    def body(*refs):
        it = iter(refs)
        src_h = next(it)
        dst_h = next(it)
        tn_h = next(it)
        wtab_h = next(it)
        if mode in (1, 2, 3):
            hp_h = next(it)
            bias_h = next(it)
        if mode in (0, 2):
            xf_h = next(it)
        hout = next(it)
        if mode == 2:
            xout = next(it)
        names = ["x0s", "x1s", "h0s", "h1s", "srcb0", "srcb1", "dstb0",
                 "dstb1", "tnb0", "tnb1", "xs0a", "xs1a", "xs0b", "xs1b",
                 "m0a", "m1a", "m0b", "m1b", "wbuf", "xb0", "xb1", "zb",
                 "lsem", "gsem", "ssem"]
        if mode in (1, 2, 3):
            names += ["bbuf", "hp0", "hp1", "hp2", "hp3"]
        if mode in (0, 2):
            names += ["fbuf"]
        sc = {n: next(it) for n in names}

        c = lax.axis_index("c")
        s = lax.axis_index("s")
        wid = s * 2 + c
        nbase = s * _TS
        iot = lax.iota(jnp.int32, 16)
        zero = jnp.zeros((16,), jnp.float32)

        pltpu.sync_copy(wtab_h, sc["wbuf"])
        if mode in (1, 2, 3):
            pltpu.sync_copy(bias_h, sc["bbuf"])
            b0v = sc["bbuf"][0, :]
            b1v = sc["bbuf"][1, :]

        for seg in range(_NSEG):
            sbase = nbase + seg * _SEG
            if mode in (1, 2, 3):
                pltpu.sync_copy(hp_h.at[pl.ds(sbase, _SEG)], sc["hp0"])
                pltpu.sync_copy(hp_h.at[pl.ds(_NP + sbase, _SEG)],
                                sc["hp1"])
                pltpu.sync_copy(hp_h.at[pl.ds(2 * _NP + sbase, _SEG)],
                                sc["hp2"])
                pltpu.sync_copy(hp_h.at[pl.ds(3 * _NP + sbase, _SEG)],
                                sc["hp3"])
            if mode in (0, 2):
                pltpu.sync_copy(xf_h.at[pl.ds(sbase * 2, 2 * _SEG)],
                                sc["fbuf"])

            def pro_body(j, _):
                off = j * 16
                if mode in (0, 2):
                    f0 = plsc.load_gather(sc["fbuf"], [(off + iot) * 2])
                    f1 = plsc.load_gather(sc["fbuf"], [(off + iot) * 2 + 1])
                if mode == 0:
                    x0v, x1v = f0, f1
                else:
                    x0v = (sc["hp0"][pl.ds(off, 16)]
                           + sc["hp2"][pl.ds(off, 16)] + b0v)
                    x1v = (sc["hp1"][pl.ds(off, 16)]
                           + sc["hp3"][pl.ds(off, 16)] + b1v)
                    x0v = jnp.maximum(x0v, 0.0)
                    x1v = jnp.maximum(x1v, 0.0)
                    if mode == 2:
                        x0v = x0v + f0
                        x1v = x1v + f1
                sc["xb0"][pl.ds(off, 16)] = x0v
                sc["xb1"][pl.ds(off, 16)] = x1v
                if seg == 0:
                    sc["zb"][pl.ds(off, 16)] = zero
                return 0

            lax.fori_loop(0, _SEG // 16, pro_body, 0)

            pltpu.sync_copy(sc["xb0"], sc["x0s"].at[pl.ds(sbase, _SEG)])
            pltpu.sync_copy(sc["xb1"], sc["x1s"].at[pl.ds(sbase, _SEG)])
            pltpu.sync_copy(sc["zb"], sc["h0s"].at[pl.ds(sbase, _SEG)])
            pltpu.sync_copy(sc["zb"], sc["h1s"].at[pl.ds(sbase, _SEG)])
            if mode == 2:
                @pl.when(c == 0)
                def _():
                    pltpu.sync_copy(sc["xb0"], xout.at[pl.ds(sbase, _SEG)])
                    pltpu.sync_copy(sc["xb1"],
                                    xout.at[pl.ds(_NP + sbase, _SEG)])
        plsc.subcore_barrier()

        srcbs = (sc["srcb0"], sc["srcb1"])
        dstbs = (sc["dstb0"], sc["dstb1"])
        tnbs = (sc["tnb0"], sc["tnb1"])
        xs0s = (sc["xs0a"], sc["xs0b"])
        xs1s = (sc["xs1a"], sc["xs1b"])
        m0s = (sc["m0a"], sc["m0b"])
        m1s = (sc["m1a"], sc["m1b"])

        def issue_linear(cidx, b):
            pltpu.async_copy(src_h.at[cidx], srcbs[b], sc["lsem"])
            pltpu.async_copy(dst_h.at[cidx], dstbs[b], sc["lsem"])
            pltpu.async_copy(tn_h.at[cidx], tnbs[b], sc["lsem"])

        def wait_linear(cidx, b):
            pltpu.make_async_copy(src_h.at[cidx], srcbs[b], sc["lsem"]).wait()
            pltpu.make_async_copy(dst_h.at[cidx], dstbs[b], sc["lsem"]).wait()
            pltpu.make_async_copy(tn_h.at[cidx], tnbs[b], sc["lsem"]).wait()

        def issue_gathers(b):
            pltpu.async_copy(sc["x0s"].at[srcbs[b]], xs0s[b], sc["gsem"])
            pltpu.async_copy(sc["x1s"].at[srcbs[b]], xs1s[b], sc["gsem"])

        def wait_gathers(b):
            pltpu.make_async_copy(sc["x0s"].at[srcbs[b]], xs0s[b],
                                  sc["gsem"]).wait()
            pltpu.make_async_copy(sc["x1s"].at[srcbs[b]], xs1s[b],
                                  sc["gsem"]).wait()

        issue_linear(wid, 0)
        wait_linear(wid, 0)
        issue_gathers(0)

        @pl.when(wid + 32 < _NCH)
        def _():
            issue_linear(wid + 32, 1)

        def chunk_pair(g, _):
            for b in (0, 1):
                k = g * 2 + b
                cidx = wid + 32 * k
                tnb = tnbs[b]
                xs0, xs1 = xs0s[b], xs1s[b]
                m0, m1 = m0s[b], m1s[b]

                @pl.when(cidx < _NCH)
                def _():
                    wait_gathers(b)
                    ncidx = cidx + 32

                    @pl.when(ncidx < _NCH)
                    def _():
                        wait_linear(ncidx, 1 - b)
                        issue_gathers(1 - b)

                    def ed_body(jj, _):
                        for sub in range(4):
                            off = jj * 64 + sub * 16
                            t = tnb[0, pl.ds(off, 16)]
                            nv = plsc.bitcast(tnb[1, pl.ds(off, 16)],
                                              jnp.float32)
                            a0 = xs0[pl.ds(off, 16)]
                            a1 = xs1[pl.ds(off, 16)]
                            w00 = plsc.load_gather(sc["wbuf"].at[0], [t])
                            w10 = plsc.load_gather(sc["wbuf"].at[1], [t])
                            w01 = plsc.load_gather(sc["wbuf"].at[2], [t])
                            w11 = plsc.load_gather(sc["wbuf"].at[3], [t])
                            na0 = a0 * nv
                            na1 = a1 * nv
                            m0[pl.ds(off, 16)] = na0 * w00 + na1 * w10
                            m1[pl.ds(off, 16)] = na0 * w01 + na1 * w11
                        return 0

                    lax.fori_loop(0, _CW // 64, ed_body, 0)
                    s0 = pltpu.async_copy(m0, sc["h0s"].at[dstbs[b]],
                                          sc["ssem"], add=True)
                    s1 = pltpu.async_copy(m1, sc["h1s"].at[dstbs[b]],
                                          sc["ssem"], add=True)
                    s0.wait()
                    s1.wait()

                    @pl.when(cidx + 64 < _NCH)
                    def _():
                        issue_linear(cidx + 64, b)
            return 0

        lax.fori_loop(0, _KMAX // 2, chunk_pair, 0)
        plsc.subcore_barrier()

        @pl.when(c == 0)
        def _():
            pltpu.sync_copy(sc["h0s"].at[pl.ds(nbase, _TS)],
                            hout.at[pl.ds(nbase, _TS)])
            pltpu.sync_copy(sc["h1s"].at[pl.ds(nbase, _TS)],
                            hout.at[pl.ds(_NP + nbase, _TS)])

        @pl.when(c == 1)
        def _():
            pltpu.sync_copy(sc["h0s"].at[pl.ds(nbase, _TS)],
                            hout.at[pl.ds(2 * _NP + nbase, _TS)])
            pltpu.sync_copy(sc["h1s"].at[pl.ds(nbase, _TS)],
                            hout.at[pl.ds(3 * _NP + nbase, _TS)])

    return pl.kernel(body, out_type=tuple(out_type) if mode == 2
                     else out_type[0],
                     compiler_params=pltpu.CompilerParams(
                         needs_layout_passes=False),
                     mesh=mesh, scratch_types=scratch)


def _mlp_body(hp_ref, x2_ref, wc_ref, b3_ref, mb_ref, out_ref):
    x40 = hp_ref[0, :] + hp_ref[2, :] + b3_ref[0, 0] + x2_ref[0, :]
    x41 = hp_ref[1, :] + hp_ref[3, :] + b3_ref[0, 1] + x2_ref[1, :]
    acc = jnp.sum(x40 * wc_ref[0, :]) + jnp.sum(x41 * wc_ref[1, :])
    z = acc + mb_ref[0, 0]
    out_ref[0, 0] = 1.0 / (1.0 + jnp.exp(-z))


def kernel(features, edge_index, edge_type, norm, bases, w_comp, layer_bias,
           mlp_w, mlp_b):
    src2 = edge_index[0].reshape(_NCH, _CW)
    dst2 = edge_index[1].reshape(_NCH, _CW)
    tn = jnp.stack(
        [edge_type.reshape(_NCH, _CW),
         jax.lax.bitcast_convert_type(norm.reshape(_NCH, _CW), jnp.int32)],
        axis=1)
    xf = jnp.pad(features.reshape(-1), (0, 2 * _NP - 2 * _N))

    W = jnp.einsum("lrb,lbio->lrio", w_comp, bases)
    wtabs = jnp.stack(
        [W[:, :, 0, 0], W[:, :, 1, 0], W[:, :, 0, 1], W[:, :, 1, 1]], axis=1
    )
    biases = jnp.broadcast_to(layer_bias[:, :, None], (_NH, _H, 16))

    k0 = _make_layer_kernel(0)
    k1 = _make_layer_kernel(1)
    k2 = _make_layer_kernel(2)
    k3 = _make_layer_kernel(3)

    h0 = k0(src2, dst2, tn, wtabs[0], xf)
    h1 = k1(src2, dst2, tn, wtabs[1], h0, biases[0])
    h2, x2 = k2(src2, dst2, tn, wtabs[2], h1, biases[1], xf)
    h3 = k3(src2, dst2, tn, wtabs[3], h2, biases[2])

    wcols = jnp.pad(mlp_w.reshape(_N, _H).T, ((0, 0), (0, _NP - _N)))
    b3 = layer_bias[3].reshape(1, _H)
    mb = mlp_b.reshape(1, 1)
    out = pl.pallas_call(
        _mlp_body,
        out_shape=jax.ShapeDtypeStruct((1, 1), jnp.float32),
        in_specs=[pl.BlockSpec(memory_space=pltpu.VMEM)] * 3
        + [pl.BlockSpec(memory_space=pltpu.SMEM)] * 2,
        out_specs=pl.BlockSpec(memory_space=pltpu.SMEM),
    )(h3.reshape(4, _NP), x2.reshape(2, _NP), wcols, b3, mb)
    return out.reshape(1, 1)
